# baseline (speedup 1.0000x reference)
import functools

import jax
import jax.numpy as jnp
from jax import lax
from jax.experimental import pallas as pl
from jax.experimental.pallas import tpu as pltpu
from jax.experimental.pallas import tpu_sc as plsc

_N = 10000
_E = 320000
_NFEAT = 128
_NHID = 64
_NCLASS = 32
_NHEADS = 4
_ALPHA = 0.2

_NC = 2
_NS = 16
_NPAD = 10240
_RT = _NPAD // _NS
_EP = 327680
_ET = _EP // (_NC * _NS)
_K = 128
_KB = _K // 128
_NCHUNK = _ET // _K
_ISLOT = 4
_RSLOT = 4
_ZR = 80


def _leaky(x):
    return jnp.where(x >= 0, x, _ALPHA * x)


def _l1_pre_body(x_ref, w_ref, av_ref, bv_ref, h_ref, as_ref, ad_ref):
    h = jnp.dot(x_ref[0], w_ref[0], preferred_element_type=jnp.float32)
    h_ref[0, pl.ds(0, _N)] = h
    h_ref[0, pl.ds(_N, _NPAD - _N)] = jnp.zeros((_NPAD - _N, _NHID),
                                                jnp.float32)
    as_ref[0, 0, pl.ds(0, _N)] = jnp.sum(h * av_ref[0, 0][None, :], axis=1)
    as_ref[0, 0, pl.ds(_N, _NPAD - _N)] = jnp.zeros((_NPAD - _N,),
                                                    jnp.float32)
    ad_ref[0, 0, pl.ds(0, _N)] = jnp.sum(h * bv_ref[0, 0][None, :], axis=1)
    ad_ref[0, 0, pl.ds(_N, _NPAD - _N)] = jnp.zeros((_NPAD - _N,),
                                                    jnp.float32)


def _l1_pre(type_emb, W, a_src, a_dst):
    return pl.pallas_call(
        _l1_pre_body,
        grid=(_NHEADS,),
        in_specs=[
            pl.BlockSpec((1, _N, _NFEAT), lambda i: (i, 0, 0)),
            pl.BlockSpec((1, _NFEAT, _NHID), lambda i: (i, 0, 0)),
            pl.BlockSpec((1, 1, _NHID), lambda i: (i, 0, 0)),
            pl.BlockSpec((1, 1, _NHID), lambda i: (i, 0, 0)),
        ],
        out_specs=[
            pl.BlockSpec((1, _NPAD, _NHID), lambda i: (i, 0, 0)),
            pl.BlockSpec((1, 1, _NPAD), lambda i: (i, 0, 0)),
            pl.BlockSpec((1, 1, _NPAD), lambda i: (i, 0, 0)),
        ],
        out_shape=[
            jax.ShapeDtypeStruct((_NHEADS, _NPAD, _NHID), jnp.float32),
            jax.ShapeDtypeStruct((_NHEADS, 1, _NPAD), jnp.float32),
            jax.ShapeDtypeStruct((_NHEADS, 1, _NPAD), jnp.float32),
        ],
    )(type_emb, W, a_src.reshape(_NHEADS, 1, _NHID),
      a_dst.reshape(_NHEADS, 1, _NHID))


def _mid_body(p_ref, d_ref, h_ref, as_ref, ad_ref, b_ref, wo_ref, ao_ref,
              bo_ref, h2_ref, as2_ref, ad2_ref):
    i = pl.program_id(0)
    es = as_ref[0, 0, pl.ds(0, _N)] + ad_ref[0, 0, pl.ds(0, _N)]
    ee = jnp.exp(_leaky(es))
    num = p_ref[0, 0] + p_ref[1, 0] + ee[:, None] * h_ref[0]
    den = (d_ref[0, 0, 0, pl.ds(0, _N)] + d_ref[1, 0, 0, pl.ds(0, _N)]
           + ee + 1e-16)
    y = _leaky(num / den[:, None] + b_ref[0, 0][None, :])
    part = jnp.dot(y, wo_ref[0], preferred_element_type=jnp.float32)

    @pl.when(i == 0)
    def _():
        h2_ref[pl.ds(0, _N)] = part
        h2_ref[pl.ds(_N, _NPAD - _N)] = jnp.zeros((_NPAD - _N, _NCLASS),
                                                  jnp.float32)

    @pl.when(i > 0)
    def _():
        h2_ref[pl.ds(0, _N)] = h2_ref[pl.ds(0, _N)] + part

    @pl.when(i == _NHEADS - 1)
    def _():
        h2 = h2_ref[pl.ds(0, _N)]
        as2_ref[0, pl.ds(0, _N)] = jnp.sum(h2 * ao_ref[0][None, :], axis=1)
        as2_ref[0, pl.ds(_N, _NPAD - _N)] = jnp.zeros((_NPAD - _N,),
                                                      jnp.float32)
        ad2_ref[0, pl.ds(0, _N)] = jnp.sum(h2 * bo_ref[0][None, :], axis=1)
        ad2_ref[0, pl.ds(_N, _NPAD - _N)] = jnp.zeros((_NPAD - _N,),
                                                      jnp.float32)


def _mid(p1, d1, h, asv, adv, b, Wo, ao, bo):
    return pl.pallas_call(
        _mid_body,
        grid=(_NHEADS,),
        in_specs=[
            pl.BlockSpec((_NC, 1, _N, _NHID), lambda i: (0, i, 0, 0)),
            pl.BlockSpec((_NC, 1, 1, _NPAD), lambda i: (0, i, 0, 0)),
            pl.BlockSpec((1, _N, _NHID), lambda i: (i, 0, 0)),
            pl.BlockSpec((1, 1, _NPAD), lambda i: (i, 0, 0)),
            pl.BlockSpec((1, 1, _NPAD), lambda i: (i, 0, 0)),
            pl.BlockSpec((1, 1, _NHID), lambda i: (i, 0, 0)),
            pl.BlockSpec((1, _NHID, _NCLASS), lambda i: (i, 0, 0)),
            pl.BlockSpec((1, _NCLASS), lambda i: (0, 0)),
            pl.BlockSpec((1, _NCLASS), lambda i: (0, 0)),
        ],
        out_specs=[
            pl.BlockSpec((_NPAD, _NCLASS), lambda i: (0, 0)),
            pl.BlockSpec((1, _NPAD), lambda i: (0, 0)),
            pl.BlockSpec((1, _NPAD), lambda i: (0, 0)),
        ],
        out_shape=[
            jax.ShapeDtypeStruct((_NPAD, _NCLASS), jnp.float32),
            jax.ShapeDtypeStruct((1, _NPAD), jnp.float32),
            jax.ShapeDtypeStruct((1, _NPAD), jnp.float32),
        ],
    )(p1, d1, h, asv, adv, b.reshape(_NHEADS, 1, _NHID), Wo, ao, bo)


def _final_body(p2_ref, d2_ref, h2_ref, as2_ref, ad2_ref, bo_ref, o_ref):
    es = as2_ref[0, pl.ds(0, _N)] + ad2_ref[0, pl.ds(0, _N)]
    ee = jnp.exp(_leaky(es))
    h2 = h2_ref[pl.ds(0, _N)]
    num = p2_ref[0, pl.ds(0, _N)] + p2_ref[1, pl.ds(0, _N)] + ee[:, None] * h2
    den = d2_ref[0, pl.ds(0, _N)] + d2_ref[1, pl.ds(0, _N)] + ee + 1e-16
    y = num / den[:, None] + bo_ref[0][None, :]
    o_ref[...] = jnp.tanh(_leaky(y))


def _final(p2, d2, h2, as2, ad2, bo):
    return pl.pallas_call(
        _final_body,
        out_shape=jax.ShapeDtypeStruct((_N, _NCLASS), jnp.float32),
    )(p2, d2, h2, as2, ad2, bo)


def _make_sc_edge(nheads, hid):
    grp = hid // 16
    mesh = plsc.VectorSubcoreMesh(
        core_axis_name="c", subcore_axis_name="s",
        num_cores=_NC, num_subcores=_NS)

    out_type = (
        jax.ShapeDtypeStruct((_NC, nheads, _NPAD, hid), jnp.float32),
        jax.ShapeDtypeStruct((_NC, nheads, _NPAD), jnp.float32),
    )
    scratch = [
        pltpu.VMEM_SHARED((_NPAD, hid), jnp.float32),
        pltpu.VMEM_SHARED((_NPAD,), jnp.float32),
        pltpu.VMEM((_ISLOT, 2 * _KB, 128), jnp.int32),
        pltpu.VMEM((_RSLOT, _K), jnp.float32),
        pltpu.VMEM((_RSLOT, _K, hid), jnp.float32),
        pltpu.VMEM((_NPAD,), jnp.float32),
        pltpu.VMEM((_NPAD,), jnp.float32),
        pltpu.VMEM((_ZR, hid), jnp.float32),
        pltpu.VMEM((_RT,), jnp.float32),
        pltpu.SemaphoreType.DMA,
        pltpu.SemaphoreType.DMA,
        pltpu.SemaphoreType.DMA,
        pltpu.SemaphoreType.DMA,
    ]

    def body(*refs):
        h_hbm = refs[0:nheads]
        as_hbm = refs[nheads:2 * nheads]
        ad_hbm = refs[2 * nheads:3 * nheads]
        edgem, out_hbm, den_hbm = refs[3 * nheads:3 * nheads + 3]
        (acc_sh, dacc_sh, idx2, ee_v, rows_v, asl, adl,
         zbuf, dzbuf, sem_idx, sem_rows, sem_scat,
         sem_io) = refs[3 * nheads + 3:]

        c = lax.axis_index("c")
        s_id = lax.axis_index("s")
        tile = c * _NS + s_id
        row0 = s_id * _RT
        rbase = (tile * _ET) // _K

        z16 = jnp.zeros((16,), jnp.float32)

        def zrow(r, carry):
            for j in range(grp):
                zbuf[r, pl.ds(j * 16, 16)] = z16
            return carry
        lax.fori_loop(0, _ZR, zrow, 0)

        def zd(r, carry):
            dzbuf[pl.ds(r * 16, 16)] = z16
            return carry
        lax.fori_loop(0, _RT // 16, zd, 0)

        def fire_idx(j, s):
            pltpu.async_copy(edgem.at[rbase + j], idx2.at[s], sem_idx)

        def wait_idx(s):
            pltpu.make_async_copy(edgem.at[rbase], idx2.at[s],
                                  sem_idx).wait()

        def fire_zero():
            for t in range(_RT // _ZR):
                pltpu.async_copy(zbuf,
                                 acc_sh.at[pl.ds(row0 + t * _ZR, _ZR)],
                                 sem_io)
            pltpu.async_copy(dzbuf, dacc_sh.at[pl.ds(row0, _RT)], sem_io)

        def wait_zero():
            for t in range(_RT // _ZR):
                pltpu.make_async_copy(
                    zbuf, acc_sh.at[pl.ds(row0 + t * _ZR, _ZR)],
                    sem_io).wait()
            pltpu.make_async_copy(dzbuf, dacc_sh.at[pl.ds(row0, _RT)],
                                  sem_io).wait()

        def fire_dump(j):
            pltpu.async_copy(acc_sh.at[pl.ds(row0, _RT)],
                             out_hbm.at[c, j, pl.ds(row0, _RT)], sem_io)
            pltpu.async_copy(dacc_sh.at[pl.ds(row0, _RT)],
                             den_hbm.at[c, j, pl.ds(row0, _RT)], sem_io)

        def wait_dump(j):
            pltpu.make_async_copy(acc_sh.at[pl.ds(row0, _RT)],
                                  out_hbm.at[c, j, pl.ds(row0, _RT)],
                                  sem_io).wait()
            pltpu.make_async_copy(dacc_sh.at[pl.ds(row0, _RT)],
                                  den_hbm.at[c, j, pl.ds(row0, _RT)],
                                  sem_io).wait()

        for i_h in range(nheads):
            pltpu.sync_copy(as_hbm[i_h], asl)
            pltpu.sync_copy(ad_hbm[i_h], adl)
            if i_h > 0:
                wait_dump(i_h - 1)
            fire_zero()

            h_i = h_hbm[i_h]

            def fire_rows(rs, ds_):
                for jb in range(_KB):
                    pltpu.async_copy(h_i.at[idx2.at[ds_, jb]],
                                     rows_v.at[rs, pl.ds(jb * 128, 128)],
                                     sem_rows)

            def wait_rows(rs, ds_):
                for jb in range(_KB):
                    pltpu.make_async_copy(
                        h_i.at[idx2.at[ds_, jb]],
                        rows_v.at[rs, pl.ds(jb * 128, 128)],
                        sem_rows).wait()

            def fire_scat(rs, ds_):
                for jb in range(_KB):
                    pltpu.async_copy(rows_v.at[rs, pl.ds(jb * 128, 128)],
                                     acc_sh.at[idx2.at[ds_, _KB + jb]],
                                     sem_scat, add=True)
                    pltpu.async_copy(ee_v.at[rs, pl.ds(jb * 128, 128)],
                                     dacc_sh.at[idx2.at[ds_, _KB + jb]],
                                     sem_scat, add=True)

            def wait_scat(rs, ds_):
                for jb in range(_KB):
                    pltpu.make_async_copy(
                        rows_v.at[rs, pl.ds(jb * 128, 128)],
                        acc_sh.at[idx2.at[ds_, _KB + jb]], sem_scat).wait()
                    pltpu.make_async_copy(
                        ee_v.at[rs, pl.ds(jb * 128, 128)],
                        dacc_sh.at[idx2.at[ds_, _KB + jb]], sem_scat).wait()

            def compute_ee(rs, ds_):
                for jb in range(_KB):
                    def eeg(g, icarry):
                        sidx = idx2[ds_, jb, pl.ds(g * 16, 16)]
                        didx = idx2[ds_, _KB + jb, pl.ds(g * 16, 16)]
                        e = (plsc.load_gather(asl, [sidx]) +
                             plsc.load_gather(adl, [didx]))
                        ee_v[rs, pl.ds(jb * 128 + g * 16, 16)] = (
                            jnp.exp(_leaky(e)))
                        return icarry
                    lax.fori_loop(0, 8, eeg, 0)

            def scale_rows(rs):
                def srow(m, icarry):
                    eev = ee_v[rs, pl.ds(m * 16, 16)]
                    base = m * 16
                    for l in range(16):
                        eek = eev[l]
                        for j in range(grp):
                            rows_v[rs, base + l, pl.ds(j * 16, 16)] = (
                                rows_v[rs, base + l, pl.ds(j * 16, 16)]
                                * eek)
                    return icarry
                lax.fori_loop(0, _K // 16, srow, 0)

            fire_idx(0, 0)
            fire_idx(1, 1)
            wait_idx(0)
            fire_rows(0, 0)
            wait_zero()
            plsc.subcore_barrier()

            def ring(p, carry):
                for q in range(_ISLOT):
                    i = p * _ISLOT + q
                    i_s = q
                    r_s = q % _RSLOT
                    n_i = (q + 1) % _ISLOT
                    n_r = (q + 1) % _RSLOT

                    @pl.when(i + 1 < _NCHUNK)
                    def _():
                        wait_idx(n_i)
                        fire_rows(n_r, n_i)

                    @pl.when(i > 0)
                    def _():
                        wait_scat((q - 1) % _RSLOT, (q - 1) % _ISLOT)

                    @pl.when(i + 2 < _NCHUNK)
                    def _():
                        fire_idx(i + 2, (q + 2) % _ISLOT)

                    wait_rows(r_s, i_s)
                    compute_ee(r_s, i_s)
                    scale_rows(r_s)
                    fire_scat(r_s, i_s)
                return carry
            lax.fori_loop(0, _NCHUNK // _ISLOT, ring, 0)
            wait_scat((_NCHUNK - 1) % _RSLOT, (_NCHUNK - 1) % _ISLOT)
            plsc.subcore_barrier()
            fire_dump(i_h)
        wait_dump(nheads - 1)

    return pl.kernel(body, out_type=out_type, mesh=mesh,
                     scratch_types=scratch,
                     compiler_params=pltpu.CompilerParams(
                         use_tc_tiling_on_sc=False,
                         needs_layout_passes=False))


_sc_edge_l1 = _make_sc_edge(_NHEADS, _NHID)
_sc_edge_l2 = _make_sc_edge(1, _NCLASS)


def kernel(type_emb, edge, W, a_src, a_dst, b, Wo, a_src_o, a_dst_o, b_o):
    src, dst = edge[0], edge[1]
    padn = _EP - _E
    fill = _N + (jnp.arange(padn, dtype=jnp.int32) % (_NPAD - _N))
    srcp = jnp.concatenate([src, fill]).reshape(_EP // _K, _KB, 128)
    dstp = jnp.concatenate([dst, fill]).reshape(_EP // _K, _KB, 128)
    edgep = jnp.concatenate([srcp, dstp], axis=1)

    h, asv, adv = _l1_pre(type_emb, W, a_src, a_dst)

    p1, d1 = _sc_edge_l1(
        h[0], h[1], h[2], h[3],
        asv[0, 0], asv[1, 0], asv[2, 0], asv[3, 0],
        adv[0, 0], adv[1, 0], adv[2, 0], adv[3, 0],
        edgep)

    h2, as2, ad2 = _mid(p1, d1.reshape(_NC, _NHEADS, 1, _NPAD), h, asv, adv,
                        b, Wo.reshape(_NHEADS, _NHID, _NCLASS),
                        a_src_o.reshape(1, -1), a_dst_o.reshape(1, -1))

    p2, d2 = _sc_edge_l2(h2, as2[0], ad2[0], edgep)

    return _final(p2[:, 0], d2[:, 0], h2, as2, ad2, b_o.reshape(1, -1))


# speedup vs baseline: 1.0655x; 1.0655x over previous
import functools

import jax
import jax.numpy as jnp
from jax import lax
from jax.experimental import pallas as pl
from jax.experimental.pallas import tpu as pltpu
from jax.experimental.pallas import tpu_sc as plsc

_N = 10000
_E = 320000
_NFEAT = 128
_NHID = 64
_NCLASS = 32
_NHEADS = 4
_ALPHA = 0.2

_NC = 2
_NS = 16
_NPAD = 10240
_RT = _NPAD // _NS
_EP = 327680
_ET = _EP // (_NC * _NS)
_K = 128
_KB = _K // 128
_NCHUNK = _ET // _K
_ISLOT = 5
_RSLOT = 5
_ZR = 80


def _leaky(x):
    return jnp.where(x >= 0, x, _ALPHA * x)


def _l1_pre_body(x_ref, w_ref, av_ref, bv_ref, h_ref, as_ref, ad_ref):
    h = jnp.dot(x_ref[0], w_ref[0], preferred_element_type=jnp.float32)
    h_ref[0, pl.ds(0, _N)] = h
    h_ref[0, pl.ds(_N, _NPAD - _N)] = jnp.zeros((_NPAD - _N, _NHID),
                                                jnp.float32)
    as_ref[0, 0, pl.ds(0, _N)] = jnp.sum(h * av_ref[0, 0][None, :], axis=1)
    as_ref[0, 0, pl.ds(_N, _NPAD - _N)] = jnp.zeros((_NPAD - _N,),
                                                    jnp.float32)
    ad_ref[0, 0, pl.ds(0, _N)] = jnp.sum(h * bv_ref[0, 0][None, :], axis=1)
    ad_ref[0, 0, pl.ds(_N, _NPAD - _N)] = jnp.zeros((_NPAD - _N,),
                                                    jnp.float32)


def _l1_pre(type_emb, W, a_src, a_dst):
    return pl.pallas_call(
        _l1_pre_body,
        grid=(_NHEADS,),
        in_specs=[
            pl.BlockSpec((1, _N, _NFEAT), lambda i: (i, 0, 0)),
            pl.BlockSpec((1, _NFEAT, _NHID), lambda i: (i, 0, 0)),
            pl.BlockSpec((1, 1, _NHID), lambda i: (i, 0, 0)),
            pl.BlockSpec((1, 1, _NHID), lambda i: (i, 0, 0)),
        ],
        out_specs=[
            pl.BlockSpec((1, _NPAD, _NHID), lambda i: (i, 0, 0)),
            pl.BlockSpec((1, 1, _NPAD), lambda i: (i, 0, 0)),
            pl.BlockSpec((1, 1, _NPAD), lambda i: (i, 0, 0)),
        ],
        out_shape=[
            jax.ShapeDtypeStruct((_NHEADS, _NPAD, _NHID), jnp.float32),
            jax.ShapeDtypeStruct((_NHEADS, 1, _NPAD), jnp.float32),
            jax.ShapeDtypeStruct((_NHEADS, 1, _NPAD), jnp.float32),
        ],
    )(type_emb, W, a_src.reshape(_NHEADS, 1, _NHID),
      a_dst.reshape(_NHEADS, 1, _NHID))


def _mid_body(p_ref, d_ref, h_ref, as_ref, ad_ref, b_ref, wo_ref, ao_ref,
              bo_ref, h2_ref, as2_ref, ad2_ref):
    i = pl.program_id(0)
    es = as_ref[0, 0, pl.ds(0, _N)] + ad_ref[0, 0, pl.ds(0, _N)]
    ee = jnp.exp(_leaky(es))
    num = p_ref[0, 0] + p_ref[1, 0] + ee[:, None] * h_ref[0]
    den = (d_ref[0, 0, 0, pl.ds(0, _N)] + d_ref[1, 0, 0, pl.ds(0, _N)]
           + ee + 1e-16)
    y = _leaky(num / den[:, None] + b_ref[0, 0][None, :])
    part = jnp.dot(y, wo_ref[0], preferred_element_type=jnp.float32)

    @pl.when(i == 0)
    def _():
        h2_ref[pl.ds(0, _N)] = part
        h2_ref[pl.ds(_N, _NPAD - _N)] = jnp.zeros((_NPAD - _N, _NCLASS),
                                                  jnp.float32)

    @pl.when(i > 0)
    def _():
        h2_ref[pl.ds(0, _N)] = h2_ref[pl.ds(0, _N)] + part

    @pl.when(i == _NHEADS - 1)
    def _():
        h2 = h2_ref[pl.ds(0, _N)]
        as2_ref[0, pl.ds(0, _N)] = jnp.sum(h2 * ao_ref[0][None, :], axis=1)
        as2_ref[0, pl.ds(_N, _NPAD - _N)] = jnp.zeros((_NPAD - _N,),
                                                      jnp.float32)
        ad2_ref[0, pl.ds(0, _N)] = jnp.sum(h2 * bo_ref[0][None, :], axis=1)
        ad2_ref[0, pl.ds(_N, _NPAD - _N)] = jnp.zeros((_NPAD - _N,),
                                                      jnp.float32)


def _mid(p1, d1, h, asv, adv, b, Wo, ao, bo):
    return pl.pallas_call(
        _mid_body,
        grid=(_NHEADS,),
        in_specs=[
            pl.BlockSpec((_NC, 1, _N, _NHID), lambda i: (0, i, 0, 0)),
            pl.BlockSpec((_NC, 1, 1, _NPAD), lambda i: (0, i, 0, 0)),
            pl.BlockSpec((1, _N, _NHID), lambda i: (i, 0, 0)),
            pl.BlockSpec((1, 1, _NPAD), lambda i: (i, 0, 0)),
            pl.BlockSpec((1, 1, _NPAD), lambda i: (i, 0, 0)),
            pl.BlockSpec((1, 1, _NHID), lambda i: (i, 0, 0)),
            pl.BlockSpec((1, _NHID, _NCLASS), lambda i: (i, 0, 0)),
            pl.BlockSpec((1, _NCLASS), lambda i: (0, 0)),
            pl.BlockSpec((1, _NCLASS), lambda i: (0, 0)),
        ],
        out_specs=[
            pl.BlockSpec((_NPAD, _NCLASS), lambda i: (0, 0)),
            pl.BlockSpec((1, _NPAD), lambda i: (0, 0)),
            pl.BlockSpec((1, _NPAD), lambda i: (0, 0)),
        ],
        out_shape=[
            jax.ShapeDtypeStruct((_NPAD, _NCLASS), jnp.float32),
            jax.ShapeDtypeStruct((1, _NPAD), jnp.float32),
            jax.ShapeDtypeStruct((1, _NPAD), jnp.float32),
        ],
    )(p1, d1, h, asv, adv, b.reshape(_NHEADS, 1, _NHID), Wo, ao, bo)


def _final_body(p2_ref, d2_ref, h2_ref, as2_ref, ad2_ref, bo_ref, o_ref):
    es = as2_ref[0, pl.ds(0, _N)] + ad2_ref[0, pl.ds(0, _N)]
    ee = jnp.exp(_leaky(es))
    h2 = h2_ref[pl.ds(0, _N)]
    num = p2_ref[0, pl.ds(0, _N)] + p2_ref[1, pl.ds(0, _N)] + ee[:, None] * h2
    den = d2_ref[0, pl.ds(0, _N)] + d2_ref[1, pl.ds(0, _N)] + ee + 1e-16
    y = num / den[:, None] + bo_ref[0][None, :]
    o_ref[...] = jnp.tanh(_leaky(y))


def _final(p2, d2, h2, as2, ad2, bo):
    return pl.pallas_call(
        _final_body,
        out_shape=jax.ShapeDtypeStruct((_N, _NCLASS), jnp.float32),
    )(p2, d2, h2, as2, ad2, bo)


def _make_sc_edge(nheads, hid):
    grp = hid // 16
    mesh = plsc.VectorSubcoreMesh(
        core_axis_name="c", subcore_axis_name="s",
        num_cores=_NC, num_subcores=_NS)

    out_type = (
        jax.ShapeDtypeStruct((_NC, nheads, _NPAD, hid), jnp.float32),
        jax.ShapeDtypeStruct((_NC, nheads, _NPAD), jnp.float32),
    )
    scratch = [
        pltpu.VMEM_SHARED((_NPAD, hid), jnp.float32),
        pltpu.VMEM_SHARED((_NPAD,), jnp.float32),
        pltpu.VMEM((_ISLOT, 2 * _KB, 128), jnp.int32),
        pltpu.VMEM((_RSLOT, _K), jnp.float32),
        pltpu.VMEM((_RSLOT, _K, hid), jnp.float32),
        pltpu.VMEM((_NPAD,), jnp.float32),
        pltpu.VMEM((_NPAD,), jnp.float32),
        pltpu.VMEM((_ZR, hid), jnp.float32),
        pltpu.VMEM((_RT,), jnp.float32),
        pltpu.SemaphoreType.DMA,
        pltpu.SemaphoreType.DMA,
        pltpu.SemaphoreType.DMA,
        pltpu.SemaphoreType.DMA,
    ]

    def body(*refs):
        h_hbm = refs[0:nheads]
        as_hbm = refs[nheads:2 * nheads]
        ad_hbm = refs[2 * nheads:3 * nheads]
        edgem, out_hbm, den_hbm = refs[3 * nheads:3 * nheads + 3]
        (acc_sh, dacc_sh, idx2, ee_v, rows_v, asl, adl,
         zbuf, dzbuf, sem_idx, sem_rows, sem_scat,
         sem_io) = refs[3 * nheads + 3:]

        c = lax.axis_index("c")
        s_id = lax.axis_index("s")
        tile = c * _NS + s_id
        row0 = s_id * _RT
        rbase = (tile * _ET) // _K

        z16 = jnp.zeros((16,), jnp.float32)

        def zrow(r, carry):
            for j in range(grp):
                zbuf[r, pl.ds(j * 16, 16)] = z16
            return carry
        lax.fori_loop(0, _ZR, zrow, 0)

        def zd(r, carry):
            dzbuf[pl.ds(r * 16, 16)] = z16
            return carry
        lax.fori_loop(0, _RT // 16, zd, 0)

        def fire_idx(j, s):
            pltpu.async_copy(edgem.at[rbase + j], idx2.at[s], sem_idx)

        def wait_idx(s):
            pltpu.make_async_copy(edgem.at[rbase], idx2.at[s],
                                  sem_idx).wait()

        def fire_zero():
            for t in range(_RT // _ZR):
                pltpu.async_copy(zbuf,
                                 acc_sh.at[pl.ds(row0 + t * _ZR, _ZR)],
                                 sem_io)
            pltpu.async_copy(dzbuf, dacc_sh.at[pl.ds(row0, _RT)], sem_io)

        def wait_zero():
            for t in range(_RT // _ZR):
                pltpu.make_async_copy(
                    zbuf, acc_sh.at[pl.ds(row0 + t * _ZR, _ZR)],
                    sem_io).wait()
            pltpu.make_async_copy(dzbuf, dacc_sh.at[pl.ds(row0, _RT)],
                                  sem_io).wait()

        def fire_dump(j):
            pltpu.async_copy(acc_sh.at[pl.ds(row0, _RT)],
                             out_hbm.at[c, j, pl.ds(row0, _RT)], sem_io)
            pltpu.async_copy(dacc_sh.at[pl.ds(row0, _RT)],
                             den_hbm.at[c, j, pl.ds(row0, _RT)], sem_io)

        def wait_dump(j):
            pltpu.make_async_copy(acc_sh.at[pl.ds(row0, _RT)],
                                  out_hbm.at[c, j, pl.ds(row0, _RT)],
                                  sem_io).wait()
            pltpu.make_async_copy(dacc_sh.at[pl.ds(row0, _RT)],
                                  den_hbm.at[c, j, pl.ds(row0, _RT)],
                                  sem_io).wait()

        for i_h in range(nheads):
            pltpu.sync_copy(as_hbm[i_h], asl)
            pltpu.sync_copy(ad_hbm[i_h], adl)
            if i_h > 0:
                wait_dump(i_h - 1)
            fire_zero()

            h_i = h_hbm[i_h]

            def fire_rows(rs, ds_):
                for jb in range(_KB):
                    pltpu.async_copy(h_i.at[idx2.at[ds_, jb]],
                                     rows_v.at[rs, pl.ds(jb * 128, 128)],
                                     sem_rows)

            def wait_rows(rs, ds_):
                for jb in range(_KB):
                    pltpu.make_async_copy(
                        h_i.at[idx2.at[ds_, jb]],
                        rows_v.at[rs, pl.ds(jb * 128, 128)],
                        sem_rows).wait()

            def fire_scat(rs, ds_):
                for jb in range(_KB):
                    pltpu.async_copy(rows_v.at[rs, pl.ds(jb * 128, 128)],
                                     acc_sh.at[idx2.at[ds_, _KB + jb]],
                                     sem_scat, add=True)
                    pltpu.async_copy(ee_v.at[rs, pl.ds(jb * 128, 128)],
                                     dacc_sh.at[idx2.at[ds_, _KB + jb]],
                                     sem_scat, add=True)

            def wait_scat(rs, ds_):
                for jb in range(_KB):
                    pltpu.make_async_copy(
                        rows_v.at[rs, pl.ds(jb * 128, 128)],
                        acc_sh.at[idx2.at[ds_, _KB + jb]], sem_scat).wait()
                    pltpu.make_async_copy(
                        ee_v.at[rs, pl.ds(jb * 128, 128)],
                        dacc_sh.at[idx2.at[ds_, _KB + jb]], sem_scat).wait()

            def compute_ee(rs, ds_):
                for jb in range(_KB):
                    def eeg(g, icarry):
                        sidx = idx2[ds_, jb, pl.ds(g * 16, 16)]
                        didx = idx2[ds_, _KB + jb, pl.ds(g * 16, 16)]
                        e = (plsc.load_gather(asl, [sidx]) +
                             plsc.load_gather(adl, [didx]))
                        ee_v[rs, pl.ds(jb * 128 + g * 16, 16)] = (
                            jnp.exp(_leaky(e)))
                        return icarry
                    lax.fori_loop(0, 8, eeg, 0)

            def scale_rows(rs):
                def srow(m, icarry):
                    eev = ee_v[rs, pl.ds(m * 16, 16)]
                    base = m * 16
                    for l in range(16):
                        eek = eev[l]
                        for j in range(grp):
                            rows_v[rs, base + l, pl.ds(j * 16, 16)] = (
                                rows_v[rs, base + l, pl.ds(j * 16, 16)]
                                * eek)
                    return icarry
                lax.fori_loop(0, _K // 16, srow, 0)

            fire_idx(0, 0)
            fire_idx(1, 1)
            wait_idx(0)
            fire_rows(0, 0)
            wait_zero()
            plsc.subcore_barrier()

            def ring(p, carry):
                for q in range(_ISLOT):
                    i = p * _ISLOT + q
                    i_s = q
                    r_s = q % _RSLOT
                    n_i = (q + 1) % _ISLOT
                    n_r = (q + 1) % _RSLOT

                    @pl.when(i + 1 < _NCHUNK)
                    def _():
                        wait_idx(n_i)
                        fire_rows(n_r, n_i)

                    @pl.when(i > 0)
                    def _():
                        wait_scat((q - 1) % _RSLOT, (q - 1) % _ISLOT)

                    @pl.when(i + 2 < _NCHUNK)
                    def _():
                        fire_idx(i + 2, (q + 2) % _ISLOT)

                    wait_rows(r_s, i_s)
                    compute_ee(r_s, i_s)
                    scale_rows(r_s)
                    fire_scat(r_s, i_s)
                return carry
            lax.fori_loop(0, _NCHUNK // _ISLOT, ring, 0)
            wait_scat((_NCHUNK - 1) % _RSLOT, (_NCHUNK - 1) % _ISLOT)
            plsc.subcore_barrier()
            fire_dump(i_h)
        wait_dump(nheads - 1)

    return pl.kernel(body, out_type=out_type, mesh=mesh,
                     scratch_types=scratch,
                     compiler_params=pltpu.CompilerParams(
                         use_tc_tiling_on_sc=False,
                         needs_layout_passes=False))


_sc_edge_l1 = _make_sc_edge(_NHEADS, _NHID)
_sc_edge_l2 = _make_sc_edge(1, _NCLASS)


def kernel(type_emb, edge, W, a_src, a_dst, b, Wo, a_src_o, a_dst_o, b_o):
    src, dst = edge[0], edge[1]
    padn = _EP - _E
    fill = _N + (jnp.arange(padn, dtype=jnp.int32) % (_NPAD - _N))
    srcp = jnp.concatenate([src, fill]).reshape(_EP // _K, _KB, 128)
    dstp = jnp.concatenate([dst, fill]).reshape(_EP // _K, _KB, 128)
    edgep = jnp.concatenate([srcp, dstp], axis=1)

    h, asv, adv = _l1_pre(type_emb, W, a_src, a_dst)

    p1, d1 = _sc_edge_l1(
        h[0], h[1], h[2], h[3],
        asv[0, 0], asv[1, 0], asv[2, 0], asv[3, 0],
        adv[0, 0], adv[1, 0], adv[2, 0], adv[3, 0],
        edgep)

    h2, as2, ad2 = _mid(p1, d1.reshape(_NC, _NHEADS, 1, _NPAD), h, asv, adv,
                        b, Wo.reshape(_NHEADS, _NHID, _NCLASS),
                        a_src_o.reshape(1, -1), a_dst_o.reshape(1, -1))

    p2, d2 = _sc_edge_l2(h2, as2[0], ad2[0], edgep)

    return _final(p2[:, 0], d2[:, 0], h2, as2, ad2, b_o.reshape(1, -1))


# speedup vs baseline: 1.4401x; 1.3515x over previous
import functools

import jax
import jax.numpy as jnp
from jax import lax
from jax.experimental import pallas as pl
from jax.experimental.pallas import tpu as pltpu
from jax.experimental.pallas import tpu_sc as plsc

_N = 10000
_E = 320000
_NFEAT = 128
_NHID = 64
_NCLASS = 32
_NHEADS = 4
_ALPHA = 0.2

_NC = 2
_NS = 16
_NPAD = 10240
_RT = _NPAD // _NS
_EP = 327680
_ET = _EP // (_NC * _NS)
_K = 128
_KB = _K // 128
_NCHUNK = _ET // _K
_ISLOT = 5
_RSLOT = 5
_ZR = 80


def _leaky(x):
    return jnp.where(x >= 0, x, _ALPHA * x)


def _l1_pre_body(x_ref, w_ref, av_ref, bv_ref, h_ref, as_ref, ad_ref):
    h = jnp.dot(x_ref[0], w_ref[0], preferred_element_type=jnp.float32)
    h_ref[0, pl.ds(0, _N)] = h
    h_ref[0, pl.ds(_N, _NPAD - _N)] = jnp.zeros((_NPAD - _N, _NHID),
                                                jnp.float32)
    as_ref[0, 0, pl.ds(0, _N)] = jnp.sum(h * av_ref[0, 0][None, :], axis=1)
    as_ref[0, 0, pl.ds(_N, _NPAD - _N)] = jnp.zeros((_NPAD - _N,),
                                                    jnp.float32)
    ad_ref[0, 0, pl.ds(0, _N)] = jnp.sum(h * bv_ref[0, 0][None, :], axis=1)
    ad_ref[0, 0, pl.ds(_N, _NPAD - _N)] = jnp.zeros((_NPAD - _N,),
                                                    jnp.float32)


def _l1_pre(type_emb, W, a_src, a_dst):
    return pl.pallas_call(
        _l1_pre_body,
        grid=(_NHEADS,),
        in_specs=[
            pl.BlockSpec((1, _N, _NFEAT), lambda i: (i, 0, 0)),
            pl.BlockSpec((1, _NFEAT, _NHID), lambda i: (i, 0, 0)),
            pl.BlockSpec((1, 1, _NHID), lambda i: (i, 0, 0)),
            pl.BlockSpec((1, 1, _NHID), lambda i: (i, 0, 0)),
        ],
        out_specs=[
            pl.BlockSpec((1, _NPAD, _NHID), lambda i: (i, 0, 0)),
            pl.BlockSpec((1, 1, _NPAD), lambda i: (i, 0, 0)),
            pl.BlockSpec((1, 1, _NPAD), lambda i: (i, 0, 0)),
        ],
        out_shape=[
            jax.ShapeDtypeStruct((_NHEADS, _NPAD, _NHID), jnp.float32),
            jax.ShapeDtypeStruct((_NHEADS, 1, _NPAD), jnp.float32),
            jax.ShapeDtypeStruct((_NHEADS, 1, _NPAD), jnp.float32),
        ],
    )(type_emb, W, a_src.reshape(_NHEADS, 1, _NHID),
      a_dst.reshape(_NHEADS, 1, _NHID))


def _mid_body(p_ref, d_ref, h_ref, as_ref, ad_ref, b_ref, wo_ref, ao_ref,
              bo_ref, h2_ref, as2_ref, ad2_ref):
    i = pl.program_id(0)
    es = as_ref[0, 0, pl.ds(0, _N)] + ad_ref[0, 0, pl.ds(0, _N)]
    ee = jnp.exp(_leaky(es))
    num = p_ref[0, 0] + p_ref[1, 0] + ee[:, None] * h_ref[0]
    den = (d_ref[0, 0, 0, pl.ds(0, _N)] + d_ref[1, 0, 0, pl.ds(0, _N)]
           + ee + 1e-16)
    y = _leaky(num / den[:, None] + b_ref[0, 0][None, :])
    part = jnp.dot(y, wo_ref[0], preferred_element_type=jnp.float32)

    @pl.when(i == 0)
    def _():
        h2_ref[pl.ds(0, _N)] = part
        h2_ref[pl.ds(_N, _NPAD - _N)] = jnp.zeros((_NPAD - _N, _NCLASS),
                                                  jnp.float32)

    @pl.when(i > 0)
    def _():
        h2_ref[pl.ds(0, _N)] = h2_ref[pl.ds(0, _N)] + part

    @pl.when(i == _NHEADS - 1)
    def _():
        h2 = h2_ref[pl.ds(0, _N)]
        as2_ref[0, pl.ds(0, _N)] = jnp.sum(h2 * ao_ref[0][None, :], axis=1)
        as2_ref[0, pl.ds(_N, _NPAD - _N)] = jnp.zeros((_NPAD - _N,),
                                                      jnp.float32)
        ad2_ref[0, pl.ds(0, _N)] = jnp.sum(h2 * bo_ref[0][None, :], axis=1)
        ad2_ref[0, pl.ds(_N, _NPAD - _N)] = jnp.zeros((_NPAD - _N,),
                                                      jnp.float32)


def _mid(p1, d1, h, asv, adv, b, Wo, ao, bo):
    return pl.pallas_call(
        _mid_body,
        grid=(_NHEADS,),
        in_specs=[
            pl.BlockSpec((_NC, 1, _N, _NHID), lambda i: (0, i, 0, 0)),
            pl.BlockSpec((_NC, 1, 1, _NPAD), lambda i: (0, i, 0, 0)),
            pl.BlockSpec((1, _N, _NHID), lambda i: (i, 0, 0)),
            pl.BlockSpec((1, 1, _NPAD), lambda i: (i, 0, 0)),
            pl.BlockSpec((1, 1, _NPAD), lambda i: (i, 0, 0)),
            pl.BlockSpec((1, 1, _NHID), lambda i: (i, 0, 0)),
            pl.BlockSpec((1, _NHID, _NCLASS), lambda i: (i, 0, 0)),
            pl.BlockSpec((1, _NCLASS), lambda i: (0, 0)),
            pl.BlockSpec((1, _NCLASS), lambda i: (0, 0)),
        ],
        out_specs=[
            pl.BlockSpec((_NPAD, _NCLASS), lambda i: (0, 0)),
            pl.BlockSpec((1, _NPAD), lambda i: (0, 0)),
            pl.BlockSpec((1, _NPAD), lambda i: (0, 0)),
        ],
        out_shape=[
            jax.ShapeDtypeStruct((_NPAD, _NCLASS), jnp.float32),
            jax.ShapeDtypeStruct((1, _NPAD), jnp.float32),
            jax.ShapeDtypeStruct((1, _NPAD), jnp.float32),
        ],
    )(p1, d1, h, asv, adv, b.reshape(_NHEADS, 1, _NHID), Wo, ao, bo)


def _final_body(p2_ref, d2_ref, h2_ref, as2_ref, ad2_ref, bo_ref, o_ref):
    es = as2_ref[0, pl.ds(0, _N)] + ad2_ref[0, pl.ds(0, _N)]
    ee = jnp.exp(_leaky(es))
    h2 = h2_ref[pl.ds(0, _N)]
    num = p2_ref[0, pl.ds(0, _N)] + p2_ref[1, pl.ds(0, _N)] + ee[:, None] * h2
    den = d2_ref[0, pl.ds(0, _N)] + d2_ref[1, pl.ds(0, _N)] + ee + 1e-16
    y = num / den[:, None] + bo_ref[0][None, :]
    o_ref[...] = jnp.tanh(_leaky(y))


def _final(p2, d2, h2, as2, ad2, bo):
    return pl.pallas_call(
        _final_body,
        out_shape=jax.ShapeDtypeStruct((_N, _NCLASS), jnp.float32),
    )(p2, d2, h2, as2, ad2, bo)


def _make_sc_edge(nheads, hid):
    grp = hid // 16
    mesh = plsc.VectorSubcoreMesh(
        core_axis_name="c", subcore_axis_name="s",
        num_cores=_NC, num_subcores=_NS)

    out_type = (
        jax.ShapeDtypeStruct((_NC, nheads, _NPAD, hid), jnp.float32),
        jax.ShapeDtypeStruct((_NC, nheads, _NPAD), jnp.float32),
    )
    scratch = [
        pltpu.VMEM_SHARED((_NPAD, hid), jnp.float32),
        pltpu.VMEM_SHARED((_NPAD,), jnp.float32),
        pltpu.VMEM((_ISLOT, 2 * _KB, 128), jnp.int32),
        pltpu.VMEM((_RSLOT, _K), jnp.float32),
        pltpu.VMEM((_RSLOT, _K, hid), jnp.float32),
        pltpu.VMEM((_NPAD,), jnp.float32),
        pltpu.VMEM((_NPAD,), jnp.float32),
        pltpu.VMEM((_ZR, hid), jnp.float32),
        pltpu.VMEM((_RT,), jnp.float32),
        pltpu.SemaphoreType.DMA,
        pltpu.SemaphoreType.DMA,
        pltpu.SemaphoreType.DMA,
        pltpu.SemaphoreType.DMA,
    ]

    def body(*refs):
        h_hbm = refs[0:nheads]
        as_hbm = refs[nheads:2 * nheads]
        ad_hbm = refs[2 * nheads:3 * nheads]
        edgem, out_hbm, den_hbm = refs[3 * nheads:3 * nheads + 3]
        (acc_sh, dacc_sh, idx2, ee_v, rows_v, asl, adl,
         zbuf, dzbuf, sem_idx, sem_rows, sem_scat,
         sem_io) = refs[3 * nheads + 3:]

        c = lax.axis_index("c")
        s_id = lax.axis_index("s")
        tile = c * _NS + s_id
        row0 = s_id * _RT
        rbase = (tile * _ET) // _K

        z16 = jnp.zeros((16,), jnp.float32)

        def zrow(r, carry):
            for j in range(grp):
                zbuf[r, pl.ds(j * 16, 16)] = z16
            return carry
        lax.fori_loop(0, _ZR, zrow, 0)

        def zd(r, carry):
            dzbuf[pl.ds(r * 16, 16)] = z16
            return carry
        lax.fori_loop(0, _RT // 16, zd, 0)

        def fire_idx(j, s):
            pltpu.async_copy(edgem.at[rbase + j], idx2.at[s], sem_idx)

        def wait_idx(s):
            pltpu.make_async_copy(edgem.at[rbase], idx2.at[s],
                                  sem_idx).wait()

        def fire_zero():
            for t in range(_RT // _ZR):
                pltpu.async_copy(zbuf,
                                 acc_sh.at[pl.ds(row0 + t * _ZR, _ZR)],
                                 sem_io)
            pltpu.async_copy(dzbuf, dacc_sh.at[pl.ds(row0, _RT)], sem_io)

        def wait_zero():
            for t in range(_RT // _ZR):
                pltpu.make_async_copy(
                    zbuf, acc_sh.at[pl.ds(row0 + t * _ZR, _ZR)],
                    sem_io).wait()
            pltpu.make_async_copy(dzbuf, dacc_sh.at[pl.ds(row0, _RT)],
                                  sem_io).wait()

        def fire_dump(j):
            pltpu.async_copy(acc_sh.at[pl.ds(row0, _RT)],
                             out_hbm.at[c, j, pl.ds(row0, _RT)], sem_io)
            pltpu.async_copy(dacc_sh.at[pl.ds(row0, _RT)],
                             den_hbm.at[c, j, pl.ds(row0, _RT)], sem_io)

        def wait_dump(j):
            pltpu.make_async_copy(acc_sh.at[pl.ds(row0, _RT)],
                                  out_hbm.at[c, j, pl.ds(row0, _RT)],
                                  sem_io).wait()
            pltpu.make_async_copy(dacc_sh.at[pl.ds(row0, _RT)],
                                  den_hbm.at[c, j, pl.ds(row0, _RT)],
                                  sem_io).wait()

        for i_h in range(nheads):
            pltpu.sync_copy(as_hbm[i_h], asl)
            pltpu.sync_copy(ad_hbm[i_h], adl)
            if i_h > 0:
                wait_dump(i_h - 1)
            fire_zero()

            h_i = h_hbm[i_h]

            def fire_rows(rs, ds_):
                for jb in range(_KB):
                    pltpu.async_copy(h_i.at[idx2.at[ds_, jb]],
                                     rows_v.at[rs, pl.ds(jb * 128, 128)],
                                     sem_rows)

            def wait_rows(rs, ds_):
                for jb in range(_KB):
                    pltpu.make_async_copy(
                        h_i.at[idx2.at[ds_, jb]],
                        rows_v.at[rs, pl.ds(jb * 128, 128)],
                        sem_rows).wait()

            def fire_scat(rs, ds_):
                for jb in range(_KB):
                    pltpu.async_copy(rows_v.at[rs, pl.ds(jb * 128, 128)],
                                     acc_sh.at[idx2.at[ds_, _KB + jb]],
                                     sem_scat, add=True)
                    pltpu.async_copy(ee_v.at[rs, pl.ds(jb * 128, 128)],
                                     dacc_sh.at[idx2.at[ds_, _KB + jb]],
                                     sem_scat, add=True)

            def wait_scat(rs, ds_):
                for jb in range(_KB):
                    pltpu.make_async_copy(
                        rows_v.at[rs, pl.ds(jb * 128, 128)],
                        acc_sh.at[idx2.at[ds_, _KB + jb]], sem_scat).wait()
                    pltpu.make_async_copy(
                        ee_v.at[rs, pl.ds(jb * 128, 128)],
                        dacc_sh.at[idx2.at[ds_, _KB + jb]], sem_scat).wait()

            def compute_ee(rs, ds_):
                for jb in range(_KB):
                    @plsc.parallel_loop(0, 8, unroll=2)
                    def _(g):
                        sidx = idx2[ds_, jb, pl.ds(g * 16, 16)]
                        didx = idx2[ds_, _KB + jb, pl.ds(g * 16, 16)]
                        e = (plsc.load_gather(asl, [sidx]) +
                             plsc.load_gather(adl, [didx]))
                        ee_v[rs, pl.ds(jb * 128 + g * 16, 16)] = (
                            jnp.exp(_leaky(e)))

            def scale_rows(rs):
                @plsc.parallel_loop(0, _K // 16, unroll=2)
                def _(m):
                    eev = ee_v[rs, pl.ds(m * 16, 16)]
                    base = m * 16
                    for l in range(16):
                        eek = eev[l]
                        for j in range(grp):
                            rows_v[rs, base + l, pl.ds(j * 16, 16)] = (
                                rows_v[rs, base + l, pl.ds(j * 16, 16)]
                                * eek)

            fire_idx(0, 0)
            fire_idx(1, 1)
            wait_idx(0)
            fire_rows(0, 0)
            wait_zero()
            plsc.subcore_barrier()

            def ring(p, carry):
                for q in range(_ISLOT):
                    i = p * _ISLOT + q
                    i_s = q
                    r_s = q % _RSLOT
                    n_i = (q + 1) % _ISLOT
                    n_r = (q + 1) % _RSLOT

                    @pl.when(i + 1 < _NCHUNK)
                    def _():
                        wait_idx(n_i)
                        fire_rows(n_r, n_i)

                    @pl.when(i > 0)
                    def _():
                        wait_scat((q - 1) % _RSLOT, (q - 1) % _ISLOT)

                    @pl.when(i + 2 < _NCHUNK)
                    def _():
                        fire_idx(i + 2, (q + 2) % _ISLOT)

                    wait_rows(r_s, i_s)
                    compute_ee(r_s, i_s)
                    scale_rows(r_s)
                    fire_scat(r_s, i_s)
                return carry
            lax.fori_loop(0, _NCHUNK // _ISLOT, ring, 0)
            wait_scat((_NCHUNK - 1) % _RSLOT, (_NCHUNK - 1) % _ISLOT)
            plsc.subcore_barrier()
            fire_dump(i_h)
        wait_dump(nheads - 1)

    return pl.kernel(body, out_type=out_type, mesh=mesh,
                     scratch_types=scratch,
                     compiler_params=pltpu.CompilerParams(
                         use_tc_tiling_on_sc=False,
                         needs_layout_passes=False))


_sc_edge_l1 = _make_sc_edge(_NHEADS, _NHID)
_sc_edge_l2 = _make_sc_edge(1, _NCLASS)


def kernel(type_emb, edge, W, a_src, a_dst, b, Wo, a_src_o, a_dst_o, b_o):
    src, dst = edge[0], edge[1]
    padn = _EP - _E
    fill = _N + (jnp.arange(padn, dtype=jnp.int32) % (_NPAD - _N))
    srcp = jnp.concatenate([src, fill]).reshape(_EP // _K, _KB, 128)
    dstp = jnp.concatenate([dst, fill]).reshape(_EP // _K, _KB, 128)
    edgep = jnp.concatenate([srcp, dstp], axis=1)

    h, asv, adv = _l1_pre(type_emb, W, a_src, a_dst)

    p1, d1 = _sc_edge_l1(
        h[0], h[1], h[2], h[3],
        asv[0, 0], asv[1, 0], asv[2, 0], asv[3, 0],
        adv[0, 0], adv[1, 0], adv[2, 0], adv[3, 0],
        edgep)

    h2, as2, ad2 = _mid(p1, d1.reshape(_NC, _NHEADS, 1, _NPAD), h, asv, adv,
                        b, Wo.reshape(_NHEADS, _NHID, _NCLASS),
                        a_src_o.reshape(1, -1), a_dst_o.reshape(1, -1))

    p2, d2 = _sc_edge_l2(h2, as2[0], ad2[0], edgep)

    return _final(p2[:, 0], d2[:, 0], h2, as2, ad2, b_o.reshape(1, -1))
